# baseline (speedup 1.0000x reference)
import jax
import jax.numpy as jnp
from jax import lax
from jax.experimental import pallas as pl
from jax.experimental.pallas import tpu as pltpu
from jax.experimental.pallas import tpu_sc as plsc

_EMB = 64
_NC = 2
_NS = 16
_NW = _NC * _NS

_SUB = 128


def _gather_body(idx_hbm, table_hbm, out_hbm,
                 idx0, idx1, rows0, rows1,
                 si0, si1, sg0, sg1, so0, so1):
  seq = idx_hbm.shape[1]
  per_w = idx_hbm.shape[0] // _NW
  wid = lax.axis_index("s") * _NC + lax.axis_index("c")
  base = wid * per_w

  splits = [(o, min(_SUB, seq - o)) for o in range(0, seq, _SUB)]
  idxs = (idx0, idx1)
  rows = (rows0, rows1)
  sem_i = (si0, si1)
  sem_g = (sg0, sg1)
  sem_o = (so0, so1)

  def fire_gathers(s):
    for o, n in splits:
      pltpu.async_copy(
          table_hbm.at[idxs[s].at[pl.ds(o, n)]],
          rows[s].at[pl.ds(o, n)],
          sem_g[s],
      )

  def wait_gathers(s):
    pltpu.make_async_copy(
        table_hbm.at[pl.ds(0, seq)], rows[s], sem_g[s]).wait()

  def wait_idx(s):
    pltpu.make_async_copy(idx_hbm.at[base], idxs[s], sem_i[s]).wait()

  def wait_store(s):
    pltpu.make_async_copy(rows[s], out_hbm.at[base], sem_o[s]).wait()

  def step(i, s):
    wait_idx(s)

    @pl.when(i >= 2)
    def _():
      wait_store(s)

    fire_gathers(s)

    @pl.when(i >= 1)
    def _():
      wait_gathers(s ^ 1)

    @pl.when(i + 1 < per_w)
    def _():
      pltpu.async_copy(idx_hbm.at[base + i + 1], idxs[s ^ 1], sem_i[s ^ 1])

    @pl.when(i >= 1)
    def _():
      pltpu.async_copy(rows[s ^ 1], out_hbm.at[base + i - 1], sem_o[s ^ 1])

  pltpu.async_copy(idx_hbm.at[base], idxs[0], sem_i[0])

  def pair(k, carry):
    step(2 * k, 0)
    step(2 * k + 1, 1)
    return carry

  lax.fori_loop(0, per_w // 2, pair, 0)

  wait_gathers(1)
  pltpu.async_copy(rows[1], out_hbm.at[base + per_w - 1], sem_o[1])
  wait_store(0)
  wait_store(1)


@jax.jit
def _embed_lookup(idx2d, table):
  b, s = idx2d.shape
  run = pl.kernel(
      _gather_body,
      out_type=jax.ShapeDtypeStruct((b, s, _EMB), jnp.float32),
      mesh=plsc.VectorSubcoreMesh(
          core_axis_name="c", subcore_axis_name="s",
          num_cores=_NC, num_subcores=_NS,
      ),
      scratch_types=[
          pltpu.VMEM((s,), jnp.int32),
          pltpu.VMEM((s,), jnp.int32),
          pltpu.VMEM((s, _EMB), jnp.float32),
          pltpu.VMEM((s, _EMB), jnp.float32),
          pltpu.SemaphoreType.DMA,
          pltpu.SemaphoreType.DMA,
          pltpu.SemaphoreType.DMA,
          pltpu.SemaphoreType.DMA,
          pltpu.SemaphoreType.DMA,
          pltpu.SemaphoreType.DMA,
      ],
      compiler_params=pltpu.CompilerParams(use_tc_tiling_on_sc=False),
  )
  return run(idx2d, table)


def kernel(input_tensor, token_table):
  return _embed_lookup(input_tensor.astype(jnp.int32), token_table)
